# Initial kernel scaffold; baseline (speedup 1.0000x reference)
import functools

import jax
import jax.numpy as jnp
from jax import lax
from jax.experimental import pallas as pl
from jax.experimental.pallas import tpu as pltpu
from jax.experimental.pallas import tpu_sc as plsc

EMBED_DIM = 64
NUM_CORES = 2
NUM_SUBCORES = 16
NUM_WORKERS = NUM_CORES * NUM_SUBCORES


@functools.lru_cache(maxsize=None)
def _make_gather(B, chunk):
  b_per_w = B // NUM_WORKERS
  n_chunks = b_per_w // chunk
  mesh = plsc.VectorSubcoreMesh(core_axis_name="c", subcore_axis_name="s")

  @functools.partial(
      pl.kernel,
      mesh=mesh,
      out_type=jax.ShapeDtypeStruct((B, EMBED_DIM), jnp.float32),
      scratch_types=[
          pltpu.VMEM((chunk,), jnp.int32),
          pltpu.VMEM((chunk, EMBED_DIM), jnp.float32),
          pltpu.SemaphoreType.DMA,
      ],
  )
  def gather_kernel(table_hbm, idx_hbm, out_hbm, idx_v, rows_v, sem):
    wid = lax.axis_index("s") * NUM_CORES + lax.axis_index("c")
    base = wid * b_per_w

    def body(i, carry):
      off = base + i * chunk
      pltpu.sync_copy(idx_hbm.at[pl.ds(off, chunk)], idx_v)
      pltpu.async_copy(table_hbm.at[idx_v], rows_v, sem).wait()
      pltpu.sync_copy(rows_v, out_hbm.at[pl.ds(off, chunk)])
      return carry

    lax.fori_loop(0, n_chunks, body, 0)

  return gather_kernel


def kernel(input_x, table):
  batch, hist = input_x.shape
  B = batch * hist
  idx = input_x.reshape(B).astype(jnp.int32)
  out = _make_gather(B, 512)(table, idx)
  emb = out.reshape(batch, hist, EMBED_DIM)
  return (emb, emb)


# speedup vs baseline: 1.7271x; 1.7271x over previous
import functools

import jax
import jax.numpy as jnp
from jax import lax
from jax.experimental import pallas as pl
from jax.experimental.pallas import tpu as pltpu
from jax.experimental.pallas import tpu_sc as plsc

EMBED_DIM = 64
NUM_CORES = 2
NUM_SUBCORES = 16
NUM_WORKERS = NUM_CORES * NUM_SUBCORES


@functools.lru_cache(maxsize=None)
def _make_gather(B, chunk):
  b_per_w = B // NUM_WORKERS
  n_chunks = b_per_w // chunk
  mesh = plsc.VectorSubcoreMesh(core_axis_name="c", subcore_axis_name="s")

  @functools.partial(
      pl.kernel,
      mesh=mesh,
      compiler_params=pltpu.CompilerParams(use_tc_tiling_on_sc=False),
      out_type=jax.ShapeDtypeStruct((B, EMBED_DIM), jnp.float32),
      scratch_types=[
          pltpu.VMEM((chunk,), jnp.int32),
          pltpu.VMEM((chunk, EMBED_DIM), jnp.float32),
          pltpu.SemaphoreType.DMA,
      ],
  )
  def gather_kernel(table_hbm, idx_hbm, out_hbm, idx_v, rows_v, sem):
    wid = lax.axis_index("s") * NUM_CORES + lax.axis_index("c")
    base = wid * b_per_w

    def body(i, carry):
      off = base + i * chunk
      pltpu.sync_copy(idx_hbm.at[pl.ds(off, chunk)], idx_v)
      pltpu.async_copy(table_hbm.at[idx_v], rows_v, sem).wait()
      pltpu.sync_copy(rows_v, out_hbm.at[pl.ds(off, chunk)])
      return carry

    lax.fori_loop(0, n_chunks, body, 0)

  return gather_kernel


def kernel(input_x, table):
  batch, hist = input_x.shape
  B = batch * hist
  idx = input_x.reshape(B).astype(jnp.int32)
  out = _make_gather(B, 512)(table, idx)
  emb = out.reshape(batch, hist, EMBED_DIM)
  return (emb, emb)


# speedup vs baseline: 1.7933x; 1.0383x over previous
import functools

import jax
import jax.numpy as jnp
from jax import lax
from jax.experimental import pallas as pl
from jax.experimental.pallas import tpu as pltpu
from jax.experimental.pallas import tpu_sc as plsc

EMBED_DIM = 64
NUM_CORES = 2
NUM_SUBCORES = 16
NUM_WORKERS = NUM_CORES * NUM_SUBCORES


@functools.lru_cache(maxsize=None)
def _make_gather(B, chunk):
  b_per_w = B // NUM_WORKERS
  n_chunks = b_per_w // chunk
  assert b_per_w % chunk == 0 and n_chunks % 2 == 0
  n2 = n_chunks // 2
  mesh = plsc.VectorSubcoreMesh(core_axis_name="c", subcore_axis_name="s")

  @functools.partial(
      pl.kernel,
      mesh=mesh,
      compiler_params=pltpu.CompilerParams(use_tc_tiling_on_sc=False),
      out_type=jax.ShapeDtypeStruct((B, EMBED_DIM), jnp.float32),
      scratch_types=[
          pltpu.VMEM((b_per_w,), jnp.int32),
          pltpu.VMEM((chunk, EMBED_DIM), jnp.float32),
          pltpu.VMEM((chunk, EMBED_DIM), jnp.float32),
          pltpu.SemaphoreType.DMA,
          pltpu.SemaphoreType.DMA,
          pltpu.SemaphoreType.DMA,
          pltpu.SemaphoreType.DMA,
      ],
  )
  def gather_kernel(table_hbm, idx_hbm, out_hbm, idx_v, rows0, rows1,
                    sem_g0, sem_g1, sem_o0, sem_o1):
    wid = lax.axis_index("s") * NUM_CORES + lax.axis_index("c")
    base = wid * b_per_w
    pltpu.sync_copy(idx_hbm.at[pl.ds(base, b_per_w)], idx_v)

    def idx_slice(c):
      return idx_v.at[pl.ds(c * chunk, chunk)]

    def start_gather(rows, sem, c):
      pltpu.async_copy(table_hbm.at[idx_slice(c)], rows, sem)

    def wait_gather(rows, sem, c):
      pltpu.make_async_copy(table_hbm.at[idx_slice(c)], rows, sem).wait()

    def start_out(rows, sem, c):
      pltpu.async_copy(rows, out_hbm.at[pl.ds(base + c * chunk, chunk)], sem)

    def wait_out(rows, sem, c):
      pltpu.make_async_copy(
          rows, out_hbm.at[pl.ds(base + c * chunk, chunk)], sem).wait()

    start_gather(rows0, sem_g0, 0)
    wait_gather(rows0, sem_g0, 0)
    start_gather(rows1, sem_g1, 1)
    start_out(rows0, sem_o0, 0)
    wait_gather(rows1, sem_g1, 1)
    wait_out(rows0, sem_o0, 0)
    start_gather(rows0, sem_g0, 2)
    start_out(rows1, sem_o1, 1)

    def body(g2, carry):
      c0 = 2 * g2
      c1 = c0 + 1
      c2 = c0 + 2
      wait_gather(rows0, sem_g0, c0)
      wait_out(rows1, sem_o1, c1 - 2)
      start_gather(rows1, sem_g1, c1)
      start_out(rows0, sem_o0, c0)
      wait_gather(rows1, sem_g1, c1)
      wait_out(rows0, sem_o0, c0)
      start_gather(rows0, sem_g0, c2)
      start_out(rows1, sem_o1, c1)
      return carry

    lax.fori_loop(1, n2 - 1, body, 0)

    c0 = n_chunks - 2
    c1 = n_chunks - 1
    wait_gather(rows0, sem_g0, c0)
    wait_out(rows1, sem_o1, c1 - 2)
    start_gather(rows1, sem_g1, c1)
    start_out(rows0, sem_o0, c0)
    wait_gather(rows1, sem_g1, c1)
    wait_out(rows0, sem_o0, c0)
    start_out(rows1, sem_o1, c1)
    wait_out(rows1, sem_o1, c1)

  return gather_kernel


def kernel(input_x, table):
  batch, hist = input_x.shape
  B = batch * hist
  idx = input_x.reshape(B).astype(jnp.int32)
  out = _make_gather(B, 512)(table, idx)
  emb = out.reshape(batch, hist, EMBED_DIM)
  return (emb, emb)
